# baseline (speedup 1.0000x reference)
import functools

import jax
import jax.numpy as jnp
from jax import lax
from jax.experimental import pallas as pl
from jax.experimental.pallas import tpu as pltpu
from jax.experimental.pallas import tpu_sc as plsc

VOCAB = 1000000
EMB = 64
BATCH = 4096
SEQ = 50

NUM_PAIRS = BATCH // 2
NC, NS, L = 2, 16, 16
NW = NC * NS
COLS_PER_W = BATCH // NW
PAIRS_PER_W = COLS_PER_W // 2

U = 524288
S = VOCAB - U
CB = 32768
GS = (S + CB - 1) // CB
VP = GS * CB
UW = U // NW
CC = 512
NCHUNK = UW // CC
DB = EMB // L


def _matmul_body(w2_ref, t_ref, out_ref):
    res = jnp.dot(w2_ref[...], t_ref[...],
                  preferred_element_type=jnp.float32)
    out_ref[pl.ds(0, CB)] = res[0]
    out_ref[pl.ds(CB, CB)] = res[1]


def _tw_lo(w2, table_t):
    return pl.pallas_call(
        _matmul_body,
        grid=(GS,),
        in_specs=[
            pl.BlockSpec((2, EMB), lambda i: (0, 0)),
            pl.BlockSpec((EMB, CB), lambda i: (0, i)),
        ],
        out_specs=pl.BlockSpec((2 * CB,), lambda i: (i,)),
        out_shape=jax.ShapeDtypeStruct((2 * VP,), jnp.float32),
    )(w2, table_t)


def _twh_body(tt_hbm, w_hbm, twh_hbm,
              w_v, tb0, tb1, ob0, ob1, sin0, sin1, sout0, sout1):
    cid = lax.axis_index("c")
    sid = lax.axis_index("s")
    wid = sid * NC + cid
    c0 = S + wid * UW

    pltpu.sync_copy(w_hbm, w_v)
    tb = (tb0, tb1)
    ob = (ob0, ob1)
    sin = (sin0, sin1)
    sout = (sout0, sout1)

    def _in_copy(c, buf):
        return pltpu.async_copy(
            tt_hbm.at[:, pl.ds(c0 + c * CC, CC)], tb[buf], sin[buf])

    _in_copy(0, 0)
    _in_copy(1, 1)

    @pl.loop(0, NCHUNK, step=2)
    def _chunks(ci):
        for buf in range(2):
            c = ci + buf
            pltpu.make_async_copy(
                tt_hbm.at[:, pl.ds(c0 + c * CC, CC)], tb[buf], sin[buf]).wait()

            @pl.when(c >= 2)
            def _():
                pltpu.make_async_copy(ob[buf].at[0], twh_hbm.at[pl.ds(0, CC)],
                                      sout[buf]).wait()
                pltpu.make_async_copy(ob[buf].at[1], twh_hbm.at[pl.ds(0, CC)],
                                      sout[buf]).wait()

            for db in range(DB):
                wv0 = w_v[pl.ds(db * L, L)]
                wv1 = w_v[pl.ds(EMB + db * L, L)]
                wb0 = [jnp.full((L,), wv0[t], jnp.float32) for t in range(L)]
                wb1 = [jnp.full((L,), wv1[t], jnp.float32) for t in range(L)]

                @pl.loop(0, CC // L)
                def _grp(g, db=db, wb0=wb0, wb1=wb1):
                    sl = pl.ds(g * L, L)
                    if db == 0:
                        a0 = jnp.zeros((L,), jnp.float32)
                        a1 = jnp.zeros((L,), jnp.float32)
                    else:
                        a0 = ob[buf][0, sl]
                        a1 = ob[buf][1, sl]
                    for t in range(L):
                        v = tb[buf][db * L + t, sl]
                        a0 = a0 + v * wb0[t]
                        a1 = a1 + v * wb1[t]
                    ob[buf][0, sl] = a0
                    ob[buf][1, sl] = a1

            base = wid * UW + c * CC
            pltpu.async_copy(ob[buf].at[0], twh_hbm.at[pl.ds(base, CC)],
                             sout[buf])
            pltpu.async_copy(ob[buf].at[1], twh_hbm.at[pl.ds(U + base, CC)],
                             sout[buf])

            @pl.when(c + 2 < NCHUNK)
            def _():
                _in_copy(c + 2, buf)

    for buf in range(2):
        pltpu.make_async_copy(ob[buf].at[0], twh_hbm.at[pl.ds(0, CC)],
                              sout[buf]).wait()
        pltpu.make_async_copy(ob[buf].at[1], twh_hbm.at[pl.ds(0, CC)],
                              sout[buf]).wait()


def _tw_hi(w_flat, table_t):
    mesh = plsc.VectorSubcoreMesh(core_axis_name="c", subcore_axis_name="s")
    return functools.partial(
        pl.kernel,
        out_type=jax.ShapeDtypeStruct((2 * U,), jnp.float32),
        mesh=mesh,
        compiler_params=pltpu.CompilerParams(
            needs_layout_passes=False, use_tc_tiling_on_sc=False),
        scratch_types=[
            pltpu.VMEM((2 * EMB,), jnp.float32),
            pltpu.VMEM((EMB, CC), jnp.float32),
            pltpu.VMEM((EMB, CC), jnp.float32),
            pltpu.VMEM((2, CC), jnp.float32),
            pltpu.VMEM((2, CC), jnp.float32),
            pltpu.SemaphoreType.DMA,
            pltpu.SemaphoreType.DMA,
            pltpu.SemaphoreType.DMA,
            pltpu.SemaphoreType.DMA,
        ],
    )(_twh_body)(table_t, w_flat)


def _sc_body(idxt_hbm, twl_hbm, twh_hbm, b_hbm, out_hbm,
             idx_v, idxa_v, idxb_v, valsa_v, valsb_v,
             colsum_v, out_v, b_v, sem):
    cid = lax.axis_index("c")
    sid = lax.axis_index("s")
    wid = sid * NC + cid
    col0 = wid * COLS_PER_W

    pltpu.sync_copy(idxt_hbm.at[:, pl.ds(col0, COLS_PER_W)], idx_v)
    pltpu.sync_copy(b_hbm, b_v)

    offl = (lax.iota(jnp.int32, L) % 2) * CB
    offh = (lax.iota(jnp.int32, L) % 2) * U
    for j in range(SEQ):
        for g in range(COLS_PER_W // L):
            sl = pl.ds(g * L, L)
            v = idx_v[j, sl]
            in_lo = v < S
            zero = jnp.zeros((L,), jnp.int32)
            idxa_v[j, sl] = jnp.where(in_lo, v + (v & jnp.int32(-CB)) + offl,
                                      zero)
            idxb_v[j, sl] = jnp.where(in_lo, zero, (v - S) + offh)

    @pl.loop(0, SEQ)
    def _fire(j):
        pltpu.async_copy(twl_hbm.at[idxa_v.at[j]], valsa_v.at[j], sem)
        pltpu.async_copy(twh_hbm.at[idxb_v.at[j]], valsb_v.at[j], sem)

    @pl.loop(0, SEQ)
    def _drain(j):
        pltpu.make_async_copy(twl_hbm.at[idxa_v.at[j]], valsa_v.at[j],
                              sem).wait()
        pltpu.make_async_copy(twh_hbm.at[idxb_v.at[j]], valsb_v.at[j],
                              sem).wait()

    for g in range(COLS_PER_W // L):
        sl = pl.ds(g * L, L)
        acc = jnp.zeros((L,), jnp.float32)
        for j in range(SEQ):
            m = idx_v[j, sl] < S
            acc = acc + jnp.where(m, valsa_v[j, sl], valsb_v[j, sl])
        colsum_v[sl] = acc

    ev = lax.iota(jnp.int32, L) * 2
    od = ev + 1
    bvec = b_v[...]
    inv = jnp.float32(1.0 / SEQ)
    for m in range(PAIRS_PER_W // L):
        base = jnp.full((L,), 2 * L * m, jnp.int32)
        evens = plsc.load_gather(colsum_v, [base + ev])
        odds = plsc.load_gather(colsum_v, [base + od])
        x = (evens + odds) * inv + bvec
        out_v[pl.ds(m * L, L)] = 1.0 / (1.0 + jnp.exp(-x))

    pltpu.sync_copy(out_v, out_hbm.at[pl.ds(wid * PAIRS_PER_W, PAIRS_PER_W)])


def _gather_pool(idx_t, twl, twh, b_vec):
    mesh = plsc.VectorSubcoreMesh(core_axis_name="c", subcore_axis_name="s")
    return functools.partial(
        pl.kernel,
        out_type=jax.ShapeDtypeStruct((NUM_PAIRS,), jnp.float32),
        mesh=mesh,
        compiler_params=pltpu.CompilerParams(
            needs_layout_passes=False, use_tc_tiling_on_sc=False),
        scratch_types=[
            pltpu.VMEM((SEQ, COLS_PER_W), jnp.int32),
            pltpu.VMEM((SEQ, COLS_PER_W), jnp.int32),
            pltpu.VMEM((SEQ, COLS_PER_W), jnp.int32),
            pltpu.VMEM((SEQ, COLS_PER_W), jnp.float32),
            pltpu.VMEM((SEQ, COLS_PER_W), jnp.float32),
            pltpu.VMEM((COLS_PER_W,), jnp.float32),
            pltpu.VMEM((PAIRS_PER_W,), jnp.float32),
            pltpu.VMEM((L,), jnp.float32),
            pltpu.SemaphoreType.DMA,
        ],
    )(_sc_body)(idx_t, twl, twh, b_vec)


@jax.jit
def _run(indices, table, W, b):
    idx_t = indices.T.astype(jnp.int32)
    table_t = table.T
    w2 = W.reshape(2, EMB)
    w_flat = W.reshape(2 * EMB)
    twl = _tw_lo(w2, table_t)
    twh = _tw_hi(w_flat, table_t)
    b_vec = jnp.broadcast_to(b.astype(jnp.float32), (L,))
    out = _gather_pool(idx_t, twl, twh, b_vec)
    return out.reshape(NUM_PAIRS, 1)


def kernel(indices, table, W, b):
    return _run(indices, table, W, b)


# speedup vs baseline: 8.6558x; 8.6558x over previous
import functools

import jax
import jax.numpy as jnp
from jax import lax
from jax.experimental import pallas as pl
from jax.experimental.pallas import tpu as pltpu
from jax.experimental.pallas import tpu_sc as plsc

VOCAB = 1000000
EMB = 64
BATCH = 4096
SEQ = 50

NUM_PAIRS = BATCH // 2
NC, NS, L = 2, 16, 16
NW = NC * NS
COLS_PER_W = BATCH // NW
PAIRS_PER_W = COLS_PER_W // 2

U = 524288
SREM = VOCAB - U
CB = 32768
GOFF = U // CB
GS = (SREM + CB - 1) // CB
VP = GS * CB
UW = U // NW
CC = 512
NCHUNK = UW // CC
DB = EMB // L


def _matmul_body(w2_ref, t_ref, out_ref):
    res = jnp.dot(w2_ref[...], t_ref[...],
                  preferred_element_type=jnp.float32)
    out_ref[pl.ds(0, CB)] = res[0]
    out_ref[pl.ds(CB, CB)] = res[1]


def _tw_lo(w2, table_t):
    return pl.pallas_call(
        _matmul_body,
        grid=(GS,),
        in_specs=[
            pl.BlockSpec((2, EMB), lambda i: (0, 0)),
            pl.BlockSpec((EMB, CB), lambda i: (0, GOFF + i)),
        ],
        out_specs=pl.BlockSpec((2 * CB,), lambda i: (i,)),
        out_shape=jax.ShapeDtypeStruct((2 * VP,), jnp.float32),
    )(w2, table_t)


def _twh_body(tt_hbm, w_hbm, twh_hbm,
              w_v, tb0, tb1, ob0, ob1, sin0, sin1, sout0, sout1):
    cid = lax.axis_index("c")
    sid = lax.axis_index("s")
    wid = sid * NC + cid
    c0 = wid * UW

    pltpu.sync_copy(w_hbm, w_v)
    tb = (tb0, tb1)
    ob = (ob0, ob1)
    sin = (sin0, sin1)
    sout = (sout0, sout1)

    def _in_copy(c, buf):
        return pltpu.async_copy(
            tt_hbm.at[:, pl.ds(c0 + c * CC, CC)], tb[buf], sin[buf])

    _in_copy(0, 0)
    _in_copy(1, 1)

    @pl.loop(0, NCHUNK, step=2)
    def _chunks(ci):
        for buf in range(2):
            c = ci + buf
            pltpu.make_async_copy(
                tt_hbm.at[:, pl.ds(c0 + c * CC, CC)], tb[buf], sin[buf]).wait()

            @pl.when(c >= 2)
            def _():
                pltpu.make_async_copy(ob[buf].at[0], twh_hbm.at[pl.ds(0, CC)],
                                      sout[buf]).wait()
                pltpu.make_async_copy(ob[buf].at[1], twh_hbm.at[pl.ds(0, CC)],
                                      sout[buf]).wait()

            for db in range(DB):
                wv0 = w_v[pl.ds(db * L, L)]
                wv1 = w_v[pl.ds(EMB + db * L, L)]
                wb0 = [jnp.full((L,), wv0[t], jnp.float32) for t in range(L)]
                wb1 = [jnp.full((L,), wv1[t], jnp.float32) for t in range(L)]

                @pl.loop(0, CC // L)
                def _grp(g, db=db, wb0=wb0, wb1=wb1):
                    sl = pl.ds(g * L, L)
                    if db == 0:
                        a0 = jnp.zeros((L,), jnp.float32)
                        a1 = jnp.zeros((L,), jnp.float32)
                    else:
                        a0 = ob[buf][0, sl]
                        a1 = ob[buf][1, sl]
                    for t in range(L):
                        v = tb[buf][db * L + t, sl]
                        a0 = a0 + v * wb0[t]
                        a1 = a1 + v * wb1[t]
                    ob[buf][0, sl] = a0
                    ob[buf][1, sl] = a1

            base = wid * UW + c * CC
            pltpu.async_copy(ob[buf].at[0], twh_hbm.at[pl.ds(base, CC)],
                             sout[buf])
            pltpu.async_copy(ob[buf].at[1], twh_hbm.at[pl.ds(U + base, CC)],
                             sout[buf])

            @pl.when(c + 2 < NCHUNK)
            def _():
                _in_copy(c + 2, buf)

    for buf in range(2):
        pltpu.make_async_copy(ob[buf].at[0], twh_hbm.at[pl.ds(0, CC)],
                              sout[buf]).wait()
        pltpu.make_async_copy(ob[buf].at[1], twh_hbm.at[pl.ds(0, CC)],
                              sout[buf]).wait()


def _tw_hi(w_flat, table_t):
    mesh = plsc.VectorSubcoreMesh(core_axis_name="c", subcore_axis_name="s")
    return functools.partial(
        pl.kernel,
        out_type=jax.ShapeDtypeStruct((2 * U,), jnp.float32),
        mesh=mesh,
        compiler_params=pltpu.CompilerParams(
            needs_layout_passes=False, use_tc_tiling_on_sc=True),
        scratch_types=[
            pltpu.VMEM((2 * EMB,), jnp.float32),
            pltpu.VMEM((EMB, CC), jnp.float32),
            pltpu.VMEM((EMB, CC), jnp.float32),
            pltpu.VMEM((2, CC), jnp.float32),
            pltpu.VMEM((2, CC), jnp.float32),
            pltpu.SemaphoreType.DMA,
            pltpu.SemaphoreType.DMA,
            pltpu.SemaphoreType.DMA,
            pltpu.SemaphoreType.DMA,
        ],
    )(_twh_body)(table_t, w_flat)


def _sc_body(idxt_hbm, twl_hbm, twh_hbm, b_hbm, out_hbm,
             idx_v, idxa_v, idxb_v, valsa_v, valsb_v,
             colsum_v, out_v, b_v, sem):
    cid = lax.axis_index("c")
    sid = lax.axis_index("s")
    wid = sid * NC + cid
    col0 = wid * COLS_PER_W

    pltpu.sync_copy(idxt_hbm.at[:, pl.ds(col0, COLS_PER_W)], idx_v)
    pltpu.sync_copy(b_hbm, b_v)

    offl = (lax.iota(jnp.int32, L) % 2) * CB
    offh = (lax.iota(jnp.int32, L) % 2) * U
    for j in range(SEQ):
        for g in range(COLS_PER_W // L):
            sl = pl.ds(g * L, L)
            v = idx_v[j, sl]
            in_hi = v < U
            vv = v - U
            zero = jnp.zeros((L,), jnp.int32)
            idxa_v[j, sl] = jnp.where(in_hi, zero,
                                      vv + (vv & jnp.int32(-CB)) + offl)
            idxb_v[j, sl] = jnp.where(in_hi, v + offh, zero)

    @pl.loop(0, SEQ)
    def _fire(j):
        pltpu.async_copy(twl_hbm.at[idxa_v.at[j]], valsa_v.at[j], sem)
        pltpu.async_copy(twh_hbm.at[idxb_v.at[j]], valsb_v.at[j], sem)

    @pl.loop(0, SEQ)
    def _drain(j):
        pltpu.make_async_copy(twl_hbm.at[idxa_v.at[j]], valsa_v.at[j],
                              sem).wait()
        pltpu.make_async_copy(twh_hbm.at[idxb_v.at[j]], valsb_v.at[j],
                              sem).wait()

    for g in range(COLS_PER_W // L):
        sl = pl.ds(g * L, L)
        acc = jnp.zeros((L,), jnp.float32)
        for j in range(SEQ):
            m = idx_v[j, sl] < U
            acc = acc + jnp.where(m, valsb_v[j, sl], valsa_v[j, sl])
        colsum_v[sl] = acc

    ev = lax.iota(jnp.int32, L) * 2
    od = ev + 1
    bvec = b_v[...]
    inv = jnp.float32(1.0 / SEQ)
    for m in range(PAIRS_PER_W // L):
        base = jnp.full((L,), 2 * L * m, jnp.int32)
        evens = plsc.load_gather(colsum_v, [base + ev])
        odds = plsc.load_gather(colsum_v, [base + od])
        x = (evens + odds) * inv + bvec
        out_v[pl.ds(m * L, L)] = 1.0 / (1.0 + jnp.exp(-x))

    pltpu.sync_copy(out_v, out_hbm.at[pl.ds(wid * PAIRS_PER_W, PAIRS_PER_W)])


def _gather_pool(idx_t, twl, twh, b_vec):
    mesh = plsc.VectorSubcoreMesh(core_axis_name="c", subcore_axis_name="s")
    return functools.partial(
        pl.kernel,
        out_type=jax.ShapeDtypeStruct((NUM_PAIRS,), jnp.float32),
        mesh=mesh,
        compiler_params=pltpu.CompilerParams(
            needs_layout_passes=False, use_tc_tiling_on_sc=False),
        scratch_types=[
            pltpu.VMEM((SEQ, COLS_PER_W), jnp.int32),
            pltpu.VMEM((SEQ, COLS_PER_W), jnp.int32),
            pltpu.VMEM((SEQ, COLS_PER_W), jnp.int32),
            pltpu.VMEM((SEQ, COLS_PER_W), jnp.float32),
            pltpu.VMEM((SEQ, COLS_PER_W), jnp.float32),
            pltpu.VMEM((COLS_PER_W,), jnp.float32),
            pltpu.VMEM((PAIRS_PER_W,), jnp.float32),
            pltpu.VMEM((L,), jnp.float32),
            pltpu.SemaphoreType.DMA,
        ],
    )(_sc_body)(idx_t, twl, twh, b_vec)


@jax.jit
def _run(indices, table, W, b):
    idx_t = indices.T.astype(jnp.int32)
    table_t = table.T
    w2 = W.reshape(2, EMB)
    w_flat = W.reshape(2 * EMB)
    twl = _tw_lo(w2, table_t)
    twh = _tw_hi(w_flat, table_t)
    b_vec = jnp.broadcast_to(b.astype(jnp.float32), (L,))
    out = _gather_pool(idx_t, twl, twh, b_vec)
    return out.reshape(NUM_PAIRS, 1)


def kernel(indices, table, W, b):
    return _run(indices, table, W, b)


# speedup vs baseline: 48.9119x; 5.6507x over previous
import functools

import jax
import jax.numpy as jnp
from jax import lax
from jax.experimental import pallas as pl
from jax.experimental.pallas import tpu as pltpu
from jax.experimental.pallas import tpu_sc as plsc

VOCAB = 1000000
EMB = 64
BATCH = 4096
SEQ = 50

NUM_PAIRS = BATCH // 2
NC, NS, L = 2, 16, 16
NW = NC * NS
COLS_PER_W = BATCH // NW
PAIRS_PER_W = COLS_PER_W // 2
CB = 32768
GS2 = (VOCAB + 2 * CB - 1) // (2 * CB)
VP = GS2 * 2 * CB


def _matmul_body(w2_ref, ta_ref, tb_ref, out_ref):
    ra = jnp.dot(w2_ref[...], ta_ref[...], preferred_element_type=jnp.float32)
    rb = jnp.dot(w2_ref[...], tb_ref[...], preferred_element_type=jnp.float32)
    out_ref[pl.ds(0, CB)] = ra[0]
    out_ref[pl.ds(CB, CB)] = ra[1]
    out_ref[pl.ds(2 * CB, CB)] = rb[0]
    out_ref[pl.ds(3 * CB, CB)] = rb[1]


def _token_weights(w2, table_t):
    return pl.pallas_call(
        _matmul_body,
        grid=(GS2,),
        in_specs=[
            pl.BlockSpec((2, EMB), lambda i: (0, 0)),
            pl.BlockSpec((EMB, CB), lambda i: (0, 2 * i)),
            pl.BlockSpec((EMB, CB),
                         lambda i: (0, jnp.minimum(2 * i + 1, 30))),
        ],
        out_specs=pl.BlockSpec((4 * CB,), lambda i: (i,)),
        out_shape=jax.ShapeDtypeStruct((2 * VP,), jnp.float32),
    )(w2, table_t, table_t)


def _sc_body(idxt_hbm, tw_hbm, b_hbm, out_hbm,
             idx_v, vals_v, colsum_v, out_v, b_v, sem):
    cid = lax.axis_index("c")
    sid = lax.axis_index("s")
    wid = sid * NC + cid
    col0 = wid * COLS_PER_W

    pltpu.sync_copy(idxt_hbm.at[:, pl.ds(col0, COLS_PER_W)], idx_v)
    pltpu.sync_copy(b_hbm, b_v)

    off = (lax.iota(jnp.int32, L) % 2) * CB
    for j in range(SEQ):
        for g in range(COLS_PER_W // L):
            sl = pl.ds(g * L, L)
            v = idx_v[j, sl]
            idx_v[j, sl] = v + (v & jnp.int32(-CB)) + off

    @pl.loop(0, SEQ)
    def _fire(j):
        pltpu.async_copy(tw_hbm.at[idx_v.at[j]], vals_v.at[j], sem)

    @pl.loop(0, SEQ)
    def _drain(j):
        pltpu.make_async_copy(tw_hbm.at[idx_v.at[j]], vals_v.at[j], sem).wait()

    for g in range(COLS_PER_W // L):
        sl = pl.ds(g * L, L)
        acc = vals_v[0, sl]
        for j in range(1, SEQ):
            acc = acc + vals_v[j, sl]
        colsum_v[sl] = acc

    ev = lax.iota(jnp.int32, L) * 2
    od = ev + 1
    bvec = b_v[...]
    inv = jnp.float32(1.0 / SEQ)
    for m in range(PAIRS_PER_W // L):
        base = jnp.full((L,), 2 * L * m, jnp.int32)
        evens = plsc.load_gather(colsum_v, [base + ev])
        odds = plsc.load_gather(colsum_v, [base + od])
        x = (evens + odds) * inv + bvec
        out_v[pl.ds(m * L, L)] = 1.0 / (1.0 + jnp.exp(-x))

    pltpu.sync_copy(out_v, out_hbm.at[pl.ds(wid * PAIRS_PER_W, PAIRS_PER_W)])


def _gather_pool(idx_t, tw_flat, b_vec):
    mesh = plsc.VectorSubcoreMesh(core_axis_name="c", subcore_axis_name="s")
    return functools.partial(
        pl.kernel,
        out_type=jax.ShapeDtypeStruct((NUM_PAIRS,), jnp.float32),
        mesh=mesh,
        compiler_params=pltpu.CompilerParams(
            needs_layout_passes=False, use_tc_tiling_on_sc=False),
        scratch_types=[
            pltpu.VMEM((SEQ, COLS_PER_W), jnp.int32),
            pltpu.VMEM((SEQ, COLS_PER_W), jnp.float32),
            pltpu.VMEM((COLS_PER_W,), jnp.float32),
            pltpu.VMEM((PAIRS_PER_W,), jnp.float32),
            pltpu.VMEM((L,), jnp.float32),
            pltpu.SemaphoreType.DMA,
        ],
    )(_sc_body)(idx_t, tw_flat, b_vec)


@jax.jit
def _run(indices, table, W, b):
    idx_t = indices.T.astype(jnp.int32)
    table_t = table.T
    w2 = W.reshape(2, EMB)
    tw = _token_weights(w2, table_t)
    b_vec = jnp.broadcast_to(b.astype(jnp.float32), (L,))
    out = _gather_pool(idx_t, tw, b_vec)
    return out.reshape(NUM_PAIRS, 1)


def kernel(indices, table, W, b):
    return _run(indices, table, W, b)
